# Initial kernel scaffold; baseline (speedup 1.0000x reference)
import functools

import jax
import jax.numpy as jnp
from jax import lax
from jax.experimental import pallas as pl
from jax.experimental.pallas import tpu as pltpu
from jax.experimental.pallas import tpu_sc as plsc

_K = 128


@functools.lru_cache(maxsize=None)
def _make_segsum(N, E, D):
    info = plsc.get_sparse_core_info()
    NC, NS = info.num_cores, info.num_subcores
    NW = NC * NS
    assert N % NS == 0
    epw = -(-E // (NW * _K)) * _K
    EP = epw * NW
    chunks = epw // _K
    rows_pw = N // NS
    NPAD = N + NS
    zr = NPAD // NS

    mesh = plsc.VectorSubcoreMesh(core_axis_name="c", subcore_axis_name="s")

    @functools.partial(
        pl.kernel,
        mesh=mesh,
        out_type=jax.ShapeDtypeStruct((NC, N, D), jnp.float32),
        scratch_types=[
            pltpu.VMEM((_K,), jnp.int32),
            pltpu.VMEM((_K,), jnp.int32),
            pltpu.VMEM((_K, D), jnp.float32),
            pltpu.VMEM_SHARED((NPAD, D), jnp.float32),
            pltpu.SemaphoreType.DMA,
        ],
    )
    def segsum(h_hbm, src_hbm, dst_hbm, zeros_hbm, out_hbm,
               sidx, didx, rows, acc, sem):
        cid = lax.axis_index("c")
        sid = lax.axis_index("s")
        wid = sid * NC + cid
        pltpu.sync_copy(zeros_hbm.at[pl.ds(sid * zr, zr), :],
                        acc.at[pl.ds(sid * zr, zr), :])
        plsc.subcore_barrier()
        base0 = wid * epw

        def body(i, carry):
            base = pl.multiple_of(base0 + i * _K, 8)
            pltpu.sync_copy(src_hbm.at[pl.ds(base, _K)], sidx)
            pltpu.sync_copy(dst_hbm.at[pl.ds(base, _K)], didx)
            pltpu.async_copy(h_hbm.at[sidx], rows, sem).wait()
            pltpu.sync_copy(rows, acc.at[didx], add=True)
            return carry

        lax.fori_loop(0, chunks, body, 0)
        plsc.subcore_barrier()
        pltpu.sync_copy(acc.at[pl.ds(sid * rows_pw, rows_pw), :],
                        out_hbm.at[cid, pl.ds(sid * rows_pw, rows_pw), :])

    return segsum, EP


def _bn(x, g, b):
    m = jnp.mean(x, axis=0, keepdims=True)
    v = jnp.mean((x - m) ** 2, axis=0, keepdims=True)
    return (x - m) / jnp.sqrt(v + 1e-5) * g + b


def _emb_body(h_ref, We_ref, be_ref, out_ref):
    out_ref[...] = jnp.dot(h_ref[...], We_ref[...],
                           preferred_element_type=jnp.float32) + be_ref[...]


def _mlp_body(h_ref, parts_ref, W1_ref, b1_ref, g1_ref, bt1_ref,
              W2_ref, b2_ref, ag_ref, ab_ref, lg_ref, lb_ref, out_ref):
    h = h_ref[...]
    z = h + parts_ref[0] + parts_ref[1]
    u = jnp.dot(z, W1_ref[...], preferred_element_type=jnp.float32) + b1_ref[...]
    t = jnp.maximum(_bn(u, g1_ref[...], bt1_ref[...]), 0.0)
    t = jnp.dot(t, W2_ref[...], preferred_element_type=jnp.float32) + b2_ref[...]
    t = jnp.maximum(_bn(t, ag_ref[...], ab_ref[...]), 0.0)
    t = _bn(t, lg_ref[...], lb_ref[...])
    t = jnp.maximum(t, 0.0)
    out_ref[...] = h + t


@functools.lru_cache(maxsize=None)
def _make_dense(N, D):
    emb = pl.pallas_call(
        _emb_body, out_shape=jax.ShapeDtypeStruct((N, D), jnp.float32))
    mlp = pl.pallas_call(
        _mlp_body, out_shape=jax.ShapeDtypeStruct((N, D), jnp.float32))
    return emb, mlp


def kernel(h, edge_index, e, We, be, mW1, mb1, mg1, mbt1, mW2, mb2,
           ag, ab, lg, lb):
    N, D = h.shape
    E = edge_index.shape[1]
    L = mW1.shape[0]
    segsum, EP = _make_segsum(N, E, D)
    emb, mlp = _make_dense(N, D)

    src = edge_index[0].astype(jnp.int32)
    dst = edge_index[1].astype(jnp.int32)
    pad = EP - E
    if pad:
        src = jnp.concatenate([src, jnp.zeros((pad,), jnp.int32)])
        dst = jnp.concatenate([dst, jnp.full((pad,), N, jnp.int32)])
    info = plsc.get_sparse_core_info()
    zeros = jnp.zeros((N + info.num_subcores, D), jnp.float32)

    r1 = lambda a: a.reshape(1, D)
    h = emb(h, We, r1(be))
    for l in range(L):
        parts = segsum(h, src, dst, zeros)
        h = mlp(h, parts, mW1[l], r1(mb1[l]), r1(mg1[l]), r1(mbt1[l]),
                mW2[l], r1(mb2[l]), r1(ag[l]), r1(ab[l]), r1(lg[l]), r1(lb[l]))
    return h


# speedup vs baseline: 3.3633x; 3.3633x over previous
import functools

import jax
import jax.numpy as jnp
from jax import lax
from jax.experimental import pallas as pl
from jax.experimental.pallas import tpu as pltpu
from jax.experimental.pallas import tpu_sc as plsc

_K = 128


@functools.lru_cache(maxsize=None)
def _make_segsum(N, E, D):
    info = plsc.get_sparse_core_info()
    NC, NS = info.num_cores, info.num_subcores
    NW = NC * NS
    assert N % NS == 0
    epw = -(-E // (NW * _K)) * _K
    EP = epw * NW
    chunks = epw // _K
    rows_pw = -(-(N + 1) // (NS * 8)) * 8
    NPAD = rows_pw * NS
    zr = rows_pw

    mesh = plsc.VectorSubcoreMesh(core_axis_name="c", subcore_axis_name="s")

    @functools.partial(
        pl.kernel,
        mesh=mesh,
        out_type=jax.ShapeDtypeStruct((NC, NPAD, D), jnp.float32),
        scratch_types=[
            pltpu.VMEM((_K,), jnp.int32),
            pltpu.VMEM((_K,), jnp.int32),
            pltpu.VMEM((_K, D), jnp.float32),
            pltpu.VMEM_SHARED((NPAD, D), jnp.float32),
            pltpu.SemaphoreType.DMA,
        ],
    )
    def segsum(h_hbm, src_hbm, dst_hbm, zeros_hbm, out_hbm,
               sidx, didx, rows, acc, sem):
        cid = lax.axis_index("c")
        sid = lax.axis_index("s")
        wid = sid * NC + cid
        pltpu.sync_copy(zeros_hbm.at[pl.ds(sid * zr, zr), :],
                        acc.at[pl.ds(sid * zr, zr), :])
        plsc.subcore_barrier()
        base0 = wid * epw

        def body(i, carry):
            base = pl.multiple_of(base0 + i * _K, 8)
            pltpu.sync_copy(src_hbm.at[pl.ds(base, _K)], sidx)
            pltpu.sync_copy(dst_hbm.at[pl.ds(base, _K)], didx)
            pltpu.async_copy(h_hbm.at[sidx], rows, sem).wait()
            pltpu.sync_copy(rows, acc.at[didx], add=True)
            return carry

        lax.fori_loop(0, chunks, body, 0)
        plsc.subcore_barrier()
        pltpu.sync_copy(acc.at[pl.ds(sid * rows_pw, rows_pw), :],
                        out_hbm.at[cid, pl.ds(sid * rows_pw, rows_pw), :])

    return segsum, EP, NPAD


def _bn(x, g, b):
    m = jnp.mean(x, axis=0, keepdims=True)
    v = jnp.mean((x - m) ** 2, axis=0, keepdims=True)
    return (x - m) / jnp.sqrt(v + 1e-5) * g + b


def _emb_body(h_ref, We_ref, be_ref, out_ref):
    out_ref[...] = jnp.dot(h_ref[...], We_ref[...],
                           preferred_element_type=jnp.float32) + be_ref[...]


def _mlp_body(h_ref, parts_ref, W1_ref, b1_ref, g1_ref, bt1_ref,
              W2_ref, b2_ref, ag_ref, ab_ref, lg_ref, lb_ref, out_ref):
    h = h_ref[...]
    n = h.shape[0]
    z = h + parts_ref[0, :n] + parts_ref[1, :n]
    u = jnp.dot(z, W1_ref[...], preferred_element_type=jnp.float32) + b1_ref[...]
    t = jnp.maximum(_bn(u, g1_ref[...], bt1_ref[...]), 0.0)
    t = jnp.dot(t, W2_ref[...], preferred_element_type=jnp.float32) + b2_ref[...]
    t = jnp.maximum(_bn(t, ag_ref[...], ab_ref[...]), 0.0)
    t = _bn(t, lg_ref[...], lb_ref[...])
    t = jnp.maximum(t, 0.0)
    out_ref[...] = h + t


@functools.lru_cache(maxsize=None)
def _make_dense(N, D):
    emb = pl.pallas_call(
        _emb_body, out_shape=jax.ShapeDtypeStruct((N, D), jnp.float32))
    mlp = pl.pallas_call(
        _mlp_body, out_shape=jax.ShapeDtypeStruct((N, D), jnp.float32))
    return emb, mlp


def kernel(h, edge_index, e, We, be, mW1, mb1, mg1, mbt1, mW2, mb2,
           ag, ab, lg, lb):
    N, D = h.shape
    E = edge_index.shape[1]
    L = mW1.shape[0]
    segsum, EP, NPAD = _make_segsum(N, E, D)
    emb, mlp = _make_dense(N, D)

    src = edge_index[0].astype(jnp.int32)
    dst = edge_index[1].astype(jnp.int32)
    pad = EP - E
    if pad:
        src = jnp.concatenate([src, jnp.zeros((pad,), jnp.int32)])
        dst = jnp.concatenate([dst, jnp.full((pad,), N, jnp.int32)])
    zeros = jnp.zeros((NPAD, D), jnp.float32)

    r1 = lambda a: a.reshape(1, D)
    h = emb(h, We, r1(be))
    for l in range(L):
        parts = segsum(h, src, dst, zeros)
        h = mlp(h, parts, mW1[l], r1(mb1[l]), r1(mg1[l]), r1(mbt1[l]),
                mW2[l], r1(mb2[l]), r1(ag[l]), r1(ab[l]), r1(lg[l]), r1(lb[l]))
    return h
